# baseline (speedup 1.0000x reference)
import functools

import jax
import jax.numpy as jnp
from jax import lax
from jax.experimental import pallas as pl
from jax.experimental.pallas import tpu as pltpu
from jax.experimental.pallas import tpu_sc as plsc

_LANES = 16
_NW = 32


def _sc_interp(idx, wl_b, wr_b, g0, g1, B, D):
  chunk = D // _NW
  nvec = chunk // _LANES
  npairs = 2 * B
  mesh = plsc.VectorSubcoreMesh(core_axis_name="c", subcore_axis_name="s")

  @functools.partial(
      pl.kernel,
      mesh=mesh,
      out_type=jax.ShapeDtypeStruct((npairs, B, D), jnp.float32),
      scratch_types=[
          pltpu.VMEM((npairs,), jnp.int32),
          pltpu.VMEM((B, _LANES), jnp.float32),
          pltpu.VMEM((B, _LANES), jnp.float32),
          pltpu.VMEM((npairs, chunk), jnp.float32),
          pltpu.VMEM((npairs, chunk), jnp.float32),
          pltpu.VMEM((2, B, chunk), jnp.float32),
          pltpu.SemaphoreType.DMA,
          pltpu.SemaphoreType.DMA,
          pltpu.SemaphoreType.DMA,
      ],
  )
  def sck(idx_hbm, wl_hbm, wr_hbm, g0_hbm, g1_hbm, out_hbm,
          idx_v, wl_v, wr_v, rows0, rows1, ob, gsem, osem0, osem1):
    wid = lax.axis_index("s") * 2 + lax.axis_index("c")
    pltpu.sync_copy(idx_hbm.at[wid], idx_v)
    pltpu.sync_copy(wl_hbm, wl_v)
    pltpu.sync_copy(wr_hbm, wr_v)
    cp0 = pltpu.async_copy(g0_hbm.at[idx_v], rows0, gsem)
    cp1 = pltpu.async_copy(g1_hbm.at[idx_v], rows1, gsem)
    cp0.wait()
    cp1.wait()

    wls = [wl_v[b2] for b2 in range(B)]
    wrs = [wr_v[b2] for b2 in range(B)]
    osems = [osem0, osem1]
    pending = [None, None]
    col = wid * chunk

    for p in range(npairs):
      b1, c = p // 2, p % 2
      rows = rows0 if c == 0 else rows1
      buf = p % 2
      if pending[buf] is not None:
        pending[buf].wait()

      def body(j, _, rows=rows, buf=buf, b1=b1):
        off = j * _LANES
        gl = rows[2 * b1, pl.ds(off, _LANES)]
        gr = rows[2 * b1 + 1, pl.ds(off, _LANES)]
        for b2 in range(B):
          ob[buf, b2, pl.ds(off, _LANES)] = wrs[b2] * gl + wls[b2] * gr
        return 0

      lax.fori_loop(0, nvec, body, 0)
      pending[buf] = pltpu.async_copy(
          ob.at[buf], out_hbm.at[p, :, pl.ds(col, chunk)], osems[buf])

    for cp in pending:
      if cp is not None:
        cp.wait()

  return sck(idx, wl_b, wr_b, g0, g1)


def kernel(patch_indices, grid0, grid1):
  T, F, H, W = grid0.shape
  B = patch_indices.shape[0]
  D = F * H * W
  chunk = D // _NW

  t = patch_indices[:, 0, 0, 0] * T
  left = jnp.floor(t).astype(jnp.int32)
  right = jnp.clip(left + 1, 0, T - 1)
  dl = t - left.astype(t.dtype)
  dr = right.astype(t.dtype) - t

  lr = jnp.stack([left, right], axis=1).reshape(-1)
  w_ids = jnp.arange(_NW, dtype=jnp.int32)[:, None]
  idx = lr[None, :] * _NW + w_ids
  wl_b = jnp.broadcast_to(dl[:, None], (B, _LANES))
  wr_b = jnp.broadcast_to(dr[:, None], (B, _LANES))

  g0 = grid0.reshape(T * _NW, chunk)
  g1 = grid1.reshape(T * _NW, chunk)
  out = _sc_interp(idx, wl_b, wr_b, g0, g1, B, D)
  return out.reshape(B, 2, B, F, H, W)
